# baseline (speedup 1.0000x reference)
import functools

import jax
import jax.numpy as jnp
from jax import lax
from jax.experimental import pallas as pl
from jax.experimental.pallas import tpu as pltpu
from jax.experimental.pallas import tpu_sc as plsc

D = 64
S = 200
NC = 2
NS = 16
NW = NC * NS
BB = 128
L = 16


def _body(x_ref, tok_ref, pos_ref, out_ref,
          idx_v, pos_v, base_v,
          gbuf0, gbuf1, sbuf0, sbuf1, oidx0, oidx1,
          gsem0, gsem1, ssem0, ssem1):
  w = lax.axis_index("s") * NC + lax.axis_index("c")

  pltpu.sync_copy(x_ref.at[w], idx_v)
  pltpu.sync_copy(pos_ref, pos_v)

  for j in range(BB // L):
    base_v[pl.ds(j * L, L)] = (lax.iota(jnp.int32, L) + (w * BB + j * L)) * S

  gbufs = (gbuf0, gbuf1)
  sbufs = (sbuf0, sbuf1)
  oidxs = (oidx0, oidx1)
  gsems = (gsem0, gsem1)
  ssems = (ssem0, ssem1)

  for b in range(2):
    pltpu.async_copy(tok_ref.at[idx_v.at[b]], gbufs[b], gsems[b])

  def step(s, gb, sb, oi, gsem, ssem):
    pltpu.make_async_copy(tok_ref.at[idx_v.at[0]], gb, gsem).wait()

    p = [pos_v[s, pl.ds(j * L, L)] for j in range(D // L)]

    @pl.when(s >= 2)
    def _():
      pltpu.make_async_copy(sb, out_ref.at[oi], ssem).wait()

    @plsc.parallel_loop(0, BB, unroll=8)
    def _(r):
      for j in range(D // L):
        sb[r, pl.ds(j * L, L)] = gb[r, pl.ds(j * L, L)] + p[j]

    for j in range(BB // L):
      oi[pl.ds(j * L, L)] = base_v[pl.ds(j * L, L)] + s

    pltpu.async_copy(sb, out_ref.at[oi], ssem)

    @pl.when(s + 2 < S)
    def _():
      pltpu.async_copy(tok_ref.at[idx_v.at[s + 2]], gb, gsem)

  @pl.loop(0, S, step=2)
  def _(s0):
    for b in range(2):
      step(s0 + b, gbufs[b], sbufs[b], oidxs[b], gsems[b], ssems[b])

  for b in range(2):
    pltpu.make_async_copy(sbufs[b], out_ref.at[oidxs[b]], ssems[b]).wait()


@jax.jit
def kernel(x, token_table, pos_table):
  bs, seq_len = x.shape
  xr = x.reshape(NW, BB, S).transpose(0, 2, 1)

  fn = pl.kernel(
      _body,
      out_type=jax.ShapeDtypeStruct((bs * S, D), jnp.float32),
      mesh=plsc.VectorSubcoreMesh(core_axis_name="c", subcore_axis_name="s"),
      compiler_params=pltpu.CompilerParams(use_tc_tiling_on_sc=False),
      scratch_types=[
          pltpu.VMEM((S, BB), jnp.int32),
          pltpu.VMEM((S, D), jnp.float32),
          pltpu.VMEM((BB,), jnp.int32),
          pltpu.VMEM((BB, D), jnp.float32),
          pltpu.VMEM((BB, D), jnp.float32),
          pltpu.VMEM((BB, D), jnp.float32),
          pltpu.VMEM((BB, D), jnp.float32),
          pltpu.VMEM((BB,), jnp.int32),
          pltpu.VMEM((BB,), jnp.int32),
          pltpu.SemaphoreType.DMA,
          pltpu.SemaphoreType.DMA,
          pltpu.SemaphoreType.DMA,
          pltpu.SemaphoreType.DMA,
      ],
  )
  out = fn(xr, token_table, pos_table)
  return out.reshape(bs, S, D)
